# baseline (speedup 1.0000x reference)
import jax
import jax.numpy as jnp
from jax import lax
from jax.experimental import pallas as pl
from jax.experimental.pallas import tpu as pltpu
from jax.experimental.pallas import tpu_sc as plsc

N_SPEAKERS = 100000
SIZE = 8192
BATCH = 4096
N_TC = 1024
N_SC = BATCH - N_TC

_info = plsc.get_sparse_core_info()
_NC = _info.num_cores
_NS = _info.num_subcores
_NW = _NC * _NS
_BPW = N_SC // _NW
_K = 2
_NBUF = 7
_G = 3
_NSTEPS = _BPW // _K


def _gather_kernel(idx_hbm, table_hbm, out_hbm, idx_v, *bufs_and_sems):
    bufs = bufs_and_sems[:_NBUF]
    gsems = bufs_and_sems[_NBUF:2 * _NBUF]
    wsems = bufs_and_sems[2 * _NBUF:3 * _NBUF]

    wid = lax.axis_index("s") * _NC + lax.axis_index("c")
    base = wid * _BPW

    pltpu.sync_copy(idx_hbm.at[wid], idx_v)

    gathers = [None] * _NBUF
    writes = [None] * _NBUF

    for j in range(_G):
        gathers[j % _NBUF] = pltpu.async_copy(
            table_hbm.at[idx_v.at[j]], bufs[j % _NBUF], gsems[j % _NBUF])

    for j in range(_NSTEPS):
        b = j % _NBUF
        gathers[b].wait()
        writes[b] = pltpu.async_copy(
            bufs[b], out_hbm.at[pl.ds(base + j * _K, _K)], wsems[b])
        nj = j + _G
        if nj < _NSTEPS:
            nb = nj % _NBUF
            if writes[nb] is not None:
                writes[nb].wait()
            gathers[nb] = pltpu.async_copy(
                table_hbm.at[idx_v.at[nj]], bufs[nb], gsems[nb])

    for b in range(_NBUF):
        if writes[b] is not None:
            writes[b].wait()


@jax.jit
def _run(sid32_sc, sid_tc, emb):
    mesh = plsc.VectorSubcoreMesh(core_axis_name="c", subcore_axis_name="s")
    out_sc = pl.kernel(
        _gather_kernel,
        mesh=mesh,
        out_type=jax.ShapeDtypeStruct((N_SC, SIZE), jnp.float32),
        scratch_types=(
            [pltpu.VMEM((_NSTEPS, _K), jnp.int32)]
            + [pltpu.VMEM((_K, SIZE), jnp.float32) for _ in range(_NBUF)]
            + [pltpu.SemaphoreType.DMA for _ in range(2 * _NBUF)]
        ),
    )(sid32_sc, emb)
    out_tc = jnp.take(emb, sid_tc, axis=0)
    return jnp.concatenate([out_tc, out_sc], axis=0)


def kernel(sid, emb):
    sid32 = sid.astype(jnp.int32)
    sid_tc = sid32[:N_TC]
    sid_sc = sid32[N_TC:].reshape(_NW, _NSTEPS, _K)
    return _run(sid_sc, sid_tc, emb)


# speedup vs baseline: 1.9346x; 1.9346x over previous
import functools

import jax
import jax.numpy as jnp
from jax import lax
from jax.experimental import pallas as pl
from jax.experimental.pallas import tpu as pltpu
from jax.experimental.pallas import tpu_sc as plsc

N_SPEAKERS = 100000
SIZE = 8192
BATCH = 4096

_info = plsc.get_sparse_core_info()
_NC = _info.num_cores
_NS = _info.num_subcores
_NW = _NC * _NS
_BPW = BATCH // _NW
_K = 1
_NBUF = 13
_G = 7
_NSTEPS = _BPW // _K


def _gather_kernel(idx_hbm, table_hbm, out_hbm, idx_v, *bufs_and_sems):
    bufs = bufs_and_sems[:_NBUF]
    gsems = bufs_and_sems[_NBUF:2 * _NBUF]
    wsems = bufs_and_sems[2 * _NBUF:3 * _NBUF]

    wid = lax.axis_index("s") * _NC + lax.axis_index("c")
    base = wid * _BPW

    pltpu.sync_copy(idx_hbm.at[wid], idx_v)

    gathers = [None] * _NBUF
    writes = [None] * _NBUF

    for j in range(_G):
        gathers[j % _NBUF] = pltpu.async_copy(
            table_hbm.at[idx_v.at[j]], bufs[j % _NBUF], gsems[j % _NBUF])

    for j in range(_NSTEPS):
        b = j % _NBUF
        gathers[b].wait()
        writes[b] = pltpu.async_copy(
            bufs[b], out_hbm.at[pl.ds(base + j * _K, _K)], wsems[b])
        nj = j + _G
        if nj < _NSTEPS:
            nb = nj % _NBUF
            if writes[nb] is not None:
                writes[nb].wait()
            gathers[nb] = pltpu.async_copy(
                table_hbm.at[idx_v.at[nj]], bufs[nb], gsems[nb])

    for b in range(_NBUF):
        if writes[b] is not None:
            writes[b].wait()


@jax.jit
def _run(sid32, emb):
    mesh = plsc.VectorSubcoreMesh(core_axis_name="c", subcore_axis_name="s")
    return pl.kernel(
        _gather_kernel,
        mesh=mesh,
        out_type=jax.ShapeDtypeStruct((BATCH, SIZE), jnp.float32),
        scratch_types=(
            [pltpu.VMEM((_NSTEPS, _K), jnp.int32)]
            + [pltpu.VMEM((_K, SIZE), jnp.float32) for _ in range(_NBUF)]
            + [pltpu.SemaphoreType.DMA for _ in range(2 * _NBUF)]
        ),
    )(sid32, emb)


def kernel(sid, emb):
    sid32 = sid.astype(jnp.int32).reshape(_NW, _NSTEPS, _K)
    return _run(sid32, emb)


# speedup vs baseline: 1.9595x; 1.0129x over previous
import functools

import jax
import jax.numpy as jnp
from jax import lax
from jax.experimental import pallas as pl
from jax.experimental.pallas import tpu as pltpu
from jax.experimental.pallas import tpu_sc as plsc

N_SPEAKERS = 100000
SIZE = 8192
BATCH = 4096

_info = plsc.get_sparse_core_info()
_NC = _info.num_cores
_NS = _info.num_subcores
_NW = _NC * _NS
_BPW = BATCH // _NW
_K = 2
_NBUF = 7
_G = 3
_NSTEPS = _BPW // _K


def _gather_kernel(idx_hbm, table_hbm, out_hbm, idx_v, *bufs_and_sems):
    bufs = bufs_and_sems[:_NBUF]
    gsems = bufs_and_sems[_NBUF:2 * _NBUF]
    wsems = bufs_and_sems[2 * _NBUF:3 * _NBUF]

    wid = lax.axis_index("s") * _NC + lax.axis_index("c")
    base = wid * _BPW

    pltpu.sync_copy(idx_hbm.at[wid], idx_v)

    gathers = [None] * _NBUF
    writes = [None] * _NBUF

    for j in range(_G):
        gathers[j % _NBUF] = pltpu.async_copy(
            table_hbm.at[idx_v.at[j]], bufs[j % _NBUF], gsems[j % _NBUF])

    for j in range(_NSTEPS):
        b = j % _NBUF
        gathers[b].wait()
        writes[b] = pltpu.async_copy(
            bufs[b], out_hbm.at[pl.ds(base + j * _K, _K)], wsems[b])
        nj = j + _G
        if nj < _NSTEPS:
            nb = nj % _NBUF
            if writes[nb] is not None:
                writes[nb].wait()
            gathers[nb] = pltpu.async_copy(
                table_hbm.at[idx_v.at[nj]], bufs[nb], gsems[nb])

    for b in range(_NBUF):
        if writes[b] is not None:
            writes[b].wait()


@jax.jit
def _run(sid32, emb):
    mesh = plsc.VectorSubcoreMesh(core_axis_name="c", subcore_axis_name="s")
    return pl.kernel(
        _gather_kernel,
        mesh=mesh,
        out_type=jax.ShapeDtypeStruct((BATCH, SIZE), jnp.float32),
        scratch_types=(
            [pltpu.VMEM((_NSTEPS, _K), jnp.int32)]
            + [pltpu.VMEM((_K, SIZE), jnp.float32) for _ in range(_NBUF)]
            + [pltpu.SemaphoreType.DMA for _ in range(2 * _NBUF)]
        ),
    )(sid32, emb)


def kernel(sid, emb):
    sid32 = sid.astype(jnp.int32).reshape(_NW, _NSTEPS, _K)
    return _run(sid32, emb)
